# baseline (speedup 1.0000x reference)
import functools
import math

import jax
import jax.numpy as jnp
from jax import lax
from jax.experimental import pallas as pl
from jax.experimental.pallas import tpu as pltpu
from jax.experimental.pallas import tpu_sc as plsc

B, L, D, H, TOPK = 2, 2048, 2048, 16, 8
DK = D // H
MT = 512

_HIGH = jax.lax.Precision.HIGHEST


def _head_mean_mat():
    d_iota = jax.lax.broadcasted_iota(jnp.int32, (D, H), 0)
    h_iota = jax.lax.broadcasted_iota(jnp.int32, (D, H), 1)
    return jnp.where(d_iota // DK == h_iota, 1.0 / DK, 0.0).astype(jnp.float32)


def _red_kernel(wq_ref, wk_ref, bq_ref, bk_ref,
                wqh_ref, wkh_ref, bqh_ref, bkh_ref):
    ones = _head_mean_mat()
    wq_bf = wq_ref[...].astype(jnp.bfloat16).astype(jnp.float32)
    wk_bf = wk_ref[...].astype(jnp.bfloat16).astype(jnp.float32)
    wqh_ref[...] = jnp.dot(wq_bf, ones, precision=_HIGH,
                           preferred_element_type=jnp.float32)
    wkh_ref[...] = jnp.dot(wk_bf, ones, precision=_HIGH,
                           preferred_element_type=jnp.float32)
    bqh_ref[...] = jnp.dot(bq_ref[...], ones, precision=_HIGH,
                           preferred_element_type=jnp.float32)
    bkh_ref[...] = jnp.dot(bk_ref[...], ones, precision=_HIGH,
                           preferred_element_type=jnp.float32)


def _means_kernel(q_ref, k_ref, wqh_ref, wkh_ref, bqh_ref, bkh_ref,
                  qm_ref, km_ref):
    q = q_ref[0].astype(jnp.bfloat16).astype(jnp.float32)
    k = k_ref[0].astype(jnp.bfloat16).astype(jnp.float32)
    qm_ref[0] = jnp.dot(q, wqh_ref[...], precision=_HIGH,
                        preferred_element_type=jnp.float32) + bqh_ref[...]
    km_ref[0] = jnp.dot(k, wkh_ref[...], precision=_HIGH,
                        preferred_element_type=jnp.float32) + bkh_ref[...]


def _spectrum_kernel(qm_ref, km_ref, c_ref, s_ref, pr_ref, pi_ref):
    qm = qm_ref[...]
    km = km_ref[...]
    C = c_ref[...]
    S = s_ref[...]
    dot = functools.partial(jnp.dot, precision=_HIGH,
                            preferred_element_type=jnp.float32)
    qC = dot(qm, C)
    qS = dot(qm, S)
    kC = dot(km, C)
    kS = dot(km, S)
    pr_ref[...] = qC * kC + qS * kS
    pi_ref[...] = qC * kS - qS * kC


def _idft_kernel(pr_ref, pi_ref, c_ref, s_ref, corr_ref):
    dot = functools.partial(jnp.dot, precision=_HIGH,
                            preferred_element_type=jnp.float32)
    corr_ref[...] = (dot(pr_ref[...], c_ref[...])
                     - dot(pi_ref[...], s_ref[...])) * (1.0 / L)


def _sc_topk_body(corr_hbm, delays_hbm, w_hbm, row_v, scrf, scri, d_v, w_v):
    wid = lax.axis_index("s") * 2 + lax.axis_index("c")
    pltpu.sync_copy(corr_hbm.at[pl.ds(wid * L, L)], row_v)
    lane = lax.iota(jnp.int32, 16)
    neg = jnp.float32(-1e30)

    def rot(scr, x, d):
        scr[pl.ds(0, 16)] = x
        scr[pl.ds(16, 16)] = x
        return scr[pl.ds(d, 16)]

    def allmax(x):
        for d in (8, 4, 2, 1):
            x = jnp.maximum(x, rot(scrf, x, d))
        return x

    def allmin_i(x):
        for d in (8, 4, 2, 1):
            x = jnp.minimum(x, rot(scri, x, d))
        return x

    def allsum(x):
        for d in (8, 4, 2, 1):
            x = x + rot(scrf, x, d)
        return x

    dvec = jnp.zeros((16,), jnp.int32)
    vvec = jnp.full((16,), neg, jnp.float32)
    prev = jnp.full((16,), -1, jnp.int32)
    for j in range(TOPK):
        def chunk_body(i, carry, prev=prev):
            bk, bv = carry
            cv = lane + i * 16
            ck = row_v[pl.ds(i * 16, 16)]
            ck = jnp.where(cv == prev, neg, ck)
            row_v[pl.ds(i * 16, 16)] = ck
            upd = ck > bk
            return (jnp.where(upd, ck, bk), jnp.where(upd, cv, bv))

        bk, bv = lax.fori_loop(
            0, L // 16, chunk_body,
            (jnp.full((16,), neg, jnp.float32), jnp.zeros((16,), jnp.int32)))
        m = allmax(bk)
        pick = allmin_i(jnp.where(bk == m, bv, jnp.int32(L)))
        dvec = jnp.where(lane == j, pick, dvec)
        vvec = jnp.where(lane == j, m, vvec)
        if j == 0:
            vmax0 = m
        prev = pick

    e = jnp.where(lane < TOPK, jnp.exp(vvec - vmax0), 0.0)
    w = e / allsum(e)
    d_v[...] = dvec
    w_v[...] = w
    pltpu.sync_copy(d_v, delays_hbm.at[pl.ds(wid * 16, 16)])
    pltpu.sync_copy(w_v, w_hbm.at[pl.ds(wid * 16, 16)])


def _topk_kernel(corr_ref, delays_ref, w_ref):
    corr = corr_ref[...]
    iota = jax.lax.broadcasted_iota(jnp.int32, corr.shape, 1)
    cur = corr
    vals, idxs = [], []
    for _ in range(TOPK):
        mx = jnp.max(cur, axis=1, keepdims=True)
        idx = jnp.min(jnp.where(cur == mx, iota, L), axis=1, keepdims=True)
        vals.append(mx)
        idxs.append(idx)
        cur = jnp.where(iota == idx, -1e30, cur)
    V = jnp.concatenate(vals, axis=1)
    I = jnp.concatenate(idxs, axis=1)
    e = jnp.exp(V - V[:, 0:1])
    w = e / jnp.sum(e, axis=1, keepdims=True)
    delays_ref[...] = I
    w_ref[...] = w


def _roll_kernel(delays_ref, wts_ref, v_ref, out_ref, v2):
    b = pl.program_id(0)
    h = pl.program_id(1)
    bh = b * H + h
    vblk = v_ref[0]
    v2[0:L, :] = vblk
    v2[L:2 * L, :] = vblk
    acc = jnp.zeros((L, DK), jnp.float32)
    for k in range(TOPK):
        d = delays_ref[bh * TOPK + k]
        w = wts_ref[bh * TOPK + k]
        acc = acc + w * v2[pl.ds(L - d, L), :]
    out_ref[0] = acc


def _proj_kernel(a_ref, w_ref, b_ref, o_ref):
    o_ref[...] = jnp.dot(a_ref[...], w_ref[...],
                         preferred_element_type=jnp.float32) + b_ref[...]


def _proj(x2d, W, bias2d):
    return pl.pallas_call(
        _proj_kernel,
        grid=(B * L // MT,),
        in_specs=[
            pl.BlockSpec((MT, D), lambda m: (m, 0)),
            pl.BlockSpec((D, D), lambda m: (0, 0)),
            pl.BlockSpec((1, D), lambda m: (0, 0)),
        ],
        out_specs=pl.BlockSpec((MT, D), lambda m: (m, 0)),
        out_shape=jax.ShapeDtypeStruct((B * L, D), jnp.float32),
    )(x2d, W, bias2d)


def kernel(queries, keys, values, Wq, bq, Wk, bk, Wv, bv, Wo, bo):
    f32 = jnp.float32
    bq2, bk2 = bq.reshape(1, D), bk.reshape(1, D)

    t = jnp.arange(L, dtype=jnp.int32)
    p = (t[:, None] * t[None, :]) % L
    ang = p.astype(f32) * f32(2.0 * math.pi / L)
    Ctab = jnp.cos(ang)
    Stab = jnp.sin(ang)

    wqh, wkh, bqh, bkh = pl.pallas_call(
        _red_kernel,
        grid=(4,),
        in_specs=[
            pl.BlockSpec((D // 4, D), lambda i: (i, 0)),
            pl.BlockSpec((D // 4, D), lambda i: (i, 0)),
            pl.BlockSpec((1, D), lambda i: (0, 0)),
            pl.BlockSpec((1, D), lambda i: (0, 0)),
        ],
        out_specs=[
            pl.BlockSpec((D // 4, H), lambda i: (i, 0)),
            pl.BlockSpec((D // 4, H), lambda i: (i, 0)),
            pl.BlockSpec((1, H), lambda i: (0, 0)),
            pl.BlockSpec((1, H), lambda i: (0, 0)),
        ],
        out_shape=[
            jax.ShapeDtypeStruct((D, H), f32),
            jax.ShapeDtypeStruct((D, H), f32),
            jax.ShapeDtypeStruct((1, H), f32),
            jax.ShapeDtypeStruct((1, H), f32),
        ],
    )(Wq, Wk, bq2, bk2)

    qm, km = pl.pallas_call(
        _means_kernel,
        grid=(B, L // MT),
        in_specs=[
            pl.BlockSpec((1, MT, D), lambda b, m: (b, m, 0)),
            pl.BlockSpec((1, MT, D), lambda b, m: (b, m, 0)),
            pl.BlockSpec((D, H), lambda b, m: (0, 0)),
            pl.BlockSpec((D, H), lambda b, m: (0, 0)),
            pl.BlockSpec((1, H), lambda b, m: (0, 0)),
            pl.BlockSpec((1, H), lambda b, m: (0, 0)),
        ],
        out_specs=[
            pl.BlockSpec((1, MT, H), lambda b, m: (b, m, 0)),
            pl.BlockSpec((1, MT, H), lambda b, m: (b, m, 0)),
        ],
        out_shape=[
            jax.ShapeDtypeStruct((B, L, H), f32),
            jax.ShapeDtypeStruct((B, L, H), f32),
        ],
    )(queries, keys, wqh, wkh, bqh, bkh)

    qmT = qm.transpose(0, 2, 1).reshape(B * H, L)
    kmT = km.transpose(0, 2, 1).reshape(B * H, L)

    FT = 512
    pr, pi = pl.pallas_call(
        _spectrum_kernel,
        grid=(L // FT,),
        in_specs=[
            pl.BlockSpec((B * H, L), lambda i: (0, 0)),
            pl.BlockSpec((B * H, L), lambda i: (0, 0)),
            pl.BlockSpec((L, FT), lambda i: (0, i)),
            pl.BlockSpec((L, FT), lambda i: (0, i)),
        ],
        out_specs=[
            pl.BlockSpec((B * H, FT), lambda i: (0, i)),
            pl.BlockSpec((B * H, FT), lambda i: (0, i)),
        ],
        out_shape=[
            jax.ShapeDtypeStruct((B * H, L), f32),
            jax.ShapeDtypeStruct((B * H, L), f32),
        ],
    )(qmT, kmT, Ctab, Stab)

    corr = pl.pallas_call(
        _idft_kernel,
        grid=(L // FT,),
        in_specs=[
            pl.BlockSpec((B * H, L), lambda i: (0, 0)),
            pl.BlockSpec((B * H, L), lambda i: (0, 0)),
            pl.BlockSpec((L, FT), lambda i: (0, i)),
            pl.BlockSpec((L, FT), lambda i: (0, i)),
        ],
        out_specs=pl.BlockSpec((B * H, FT), lambda i: (0, i)),
        out_shape=jax.ShapeDtypeStruct((B * H, L), f32),
    )(pr, pi, Ctab, Stab)

    sc_topk = pl.kernel(
        _sc_topk_body,
        out_type=[
            jax.ShapeDtypeStruct((B * H * 16,), jnp.int32),
            jax.ShapeDtypeStruct((B * H * 16,), f32),
        ],
        mesh=plsc.VectorSubcoreMesh(core_axis_name="c", subcore_axis_name="s"),
        scratch_types=[
            pltpu.VMEM((L,), f32),
            pltpu.VMEM((32,), f32),
            pltpu.VMEM((32,), jnp.int32),
            pltpu.VMEM((16,), jnp.int32),
            pltpu.VMEM((16,), f32),
        ],
    )
    d16, w16 = sc_topk(corr.reshape(-1))
    delays = d16.reshape(B * H, 16)[:, :TOPK]
    wts = w16.reshape(B * H, 16)[:, :TOPK]

    v = _proj(values.reshape(B * L, D), Wv, bv.reshape(1, D)).reshape(B, L, D)

    ctx = pl.pallas_call(
        _roll_kernel,
        grid=(B, H),
        in_specs=[
            pl.BlockSpec(memory_space=pltpu.SMEM),
            pl.BlockSpec(memory_space=pltpu.SMEM),
            pl.BlockSpec((1, L, DK), lambda b, h: (b, 0, h)),
        ],
        out_specs=pl.BlockSpec((1, L, DK), lambda b, h: (b, 0, h)),
        out_shape=jax.ShapeDtypeStruct((B, L, D), f32),
        scratch_shapes=[pltpu.VMEM((2 * L, DK), f32)],
    )(delays.reshape(-1), wts.reshape(-1), v)

    out = _proj(ctx.reshape(B * L, D), Wo, bo.reshape(1, D))
    return out.reshape(B, L, D)


# speedup vs baseline: 1.1593x; 1.1593x over previous
import functools
import math

import jax
import jax.numpy as jnp
from jax import lax
from jax.experimental import pallas as pl
from jax.experimental.pallas import tpu as pltpu
from jax.experimental.pallas import tpu_sc as plsc

B, L, D, H, TOPK = 2, 2048, 2048, 16, 8
DK = D // H
MT = 512

_HIGH = jax.lax.Precision.HIGHEST


def _head_mean_mat():
    d_iota = jax.lax.broadcasted_iota(jnp.int32, (D, H), 0)
    h_iota = jax.lax.broadcasted_iota(jnp.int32, (D, H), 1)
    return jnp.where(d_iota // DK == h_iota, 1.0 / DK, 0.0).astype(jnp.float32)


def _red_kernel(wq_ref, wk_ref, bq_ref, bk_ref,
                wqh_ref, wkh_ref, bqh_ref, bkh_ref):
    ones = _head_mean_mat()
    wq_bf = wq_ref[...].astype(jnp.bfloat16).astype(jnp.float32)
    wk_bf = wk_ref[...].astype(jnp.bfloat16).astype(jnp.float32)
    wqh_ref[...] = jnp.dot(wq_bf, ones, preferred_element_type=jnp.float32)
    wkh_ref[...] = jnp.dot(wk_bf, ones, preferred_element_type=jnp.float32)
    bqh_ref[...] = jnp.dot(bq_ref[...], ones, precision=_HIGH,
                           preferred_element_type=jnp.float32)
    bkh_ref[...] = jnp.dot(bk_ref[...], ones, precision=_HIGH,
                           preferred_element_type=jnp.float32)


def _means_kernel(q_ref, k_ref, wqh_ref, wkh_ref, bqh_ref, bkh_ref,
                  qm_ref, km_ref):
    def split3_dot(x, w):
        acc = None
        rem = w
        for _ in range(3):
            piece = rem.astype(jnp.bfloat16).astype(jnp.float32)
            rem = rem - piece
            t = jnp.dot(x, piece, preferred_element_type=jnp.float32)
            acc = t if acc is None else acc + t
        return acc

    q = q_ref[0].astype(jnp.bfloat16).astype(jnp.float32)
    k = k_ref[0].astype(jnp.bfloat16).astype(jnp.float32)
    qm_ref[0] = split3_dot(q, wqh_ref[...]) + bqh_ref[...]
    km_ref[0] = split3_dot(k, wkh_ref[...]) + bkh_ref[...]


def _spectrum_kernel(qm_ref, km_ref, c_ref, s_ref, pr_ref, pi_ref):
    qm = qm_ref[...]
    km = km_ref[...]
    C = c_ref[...]
    S = s_ref[...]
    dot = functools.partial(jnp.dot, precision=_HIGH,
                            preferred_element_type=jnp.float32)
    qC = dot(qm, C)
    qS = dot(qm, S)
    kC = dot(km, C)
    kS = dot(km, S)
    pr_ref[...] = qC * kC + qS * kS
    pi_ref[...] = qC * kS - qS * kC


def _idft_kernel(pr_ref, pi_ref, c_ref, s_ref, corr_ref):
    dot = functools.partial(jnp.dot, precision=_HIGH,
                            preferred_element_type=jnp.float32)
    corr_ref[...] = (dot(pr_ref[...], c_ref[...])
                     - dot(pi_ref[...], s_ref[...])) * (1.0 / L)


def _sc_topk_body(corr_hbm, delays_hbm, w_hbm, row_v, scrf, scri, d_v, w_v):
    wid = lax.axis_index("s") * 2 + lax.axis_index("c")
    pltpu.sync_copy(corr_hbm.at[pl.ds(wid * L, L)], row_v)
    lane = lax.iota(jnp.int32, 16)
    neg = jnp.float32(-1e30)

    def rot(scr, x, d):
        scr[pl.ds(0, 16)] = x
        scr[pl.ds(16, 16)] = x
        return scr[pl.ds(d, 16)]

    def allmax(x):
        for d in (8, 4, 2, 1):
            x = jnp.maximum(x, rot(scrf, x, d))
        return x

    def allmin_i(x):
        for d in (8, 4, 2, 1):
            x = jnp.minimum(x, rot(scri, x, d))
        return x

    def allsum(x):
        for d in (8, 4, 2, 1):
            x = x + rot(scrf, x, d)
        return x

    dvec = jnp.zeros((16,), jnp.int32)
    vvec = jnp.full((16,), neg, jnp.float32)
    prev = jnp.full((16,), -1, jnp.int32)
    for j in range(TOPK):
        def chunk_body(i, carry, prev=prev):
            bk, bv = carry
            cv = lane + i * 16
            ck = row_v[pl.ds(i * 16, 16)]
            ck = jnp.where(cv == prev, neg, ck)
            row_v[pl.ds(i * 16, 16)] = ck
            upd = ck > bk
            return (jnp.where(upd, ck, bk), jnp.where(upd, cv, bv))

        bk, bv = lax.fori_loop(
            0, L // 16, chunk_body,
            (jnp.full((16,), neg, jnp.float32), jnp.zeros((16,), jnp.int32)))
        m = allmax(bk)
        pick = allmin_i(jnp.where(bk == m, bv, jnp.int32(L)))
        dvec = jnp.where(lane == j, pick, dvec)
        vvec = jnp.where(lane == j, m, vvec)
        if j == 0:
            vmax0 = m
        prev = pick

    e = jnp.where(lane < TOPK, jnp.exp(vvec - vmax0), 0.0)
    w = e / allsum(e)
    d_v[...] = dvec
    w_v[...] = w
    pltpu.sync_copy(d_v, delays_hbm.at[pl.ds(wid * 16, 16)])
    pltpu.sync_copy(w_v, w_hbm.at[pl.ds(wid * 16, 16)])


def _topk_kernel(corr_ref, delays_ref, w_ref):
    corr = corr_ref[...]
    iota = jax.lax.broadcasted_iota(jnp.int32, corr.shape, 1)
    cur = corr
    vals, idxs = [], []
    for _ in range(TOPK):
        mx = jnp.max(cur, axis=1, keepdims=True)
        idx = jnp.min(jnp.where(cur == mx, iota, L), axis=1, keepdims=True)
        vals.append(mx)
        idxs.append(idx)
        cur = jnp.where(iota == idx, -1e30, cur)
    V = jnp.concatenate(vals, axis=1)
    I = jnp.concatenate(idxs, axis=1)
    e = jnp.exp(V - V[:, 0:1])
    w = e / jnp.sum(e, axis=1, keepdims=True)
    delays_ref[...] = I
    w_ref[...] = w


def _roll_kernel(delays_ref, wts_ref, v_ref, out_ref, v2):
    b = pl.program_id(0)
    h = pl.program_id(1)
    bh = b * H + h
    vblk = v_ref[0]
    v2[0:L, :] = vblk
    v2[L:2 * L, :] = vblk
    acc = jnp.zeros((L, DK), jnp.float32)
    for k in range(TOPK):
        d = delays_ref[bh * TOPK + k]
        w = wts_ref[bh * TOPK + k]
        acc = acc + w * v2[pl.ds(L - d, L), :]
    out_ref[0] = acc


def _proj_kernel(a_ref, w_ref, b_ref, o_ref):
    o_ref[...] = jnp.dot(a_ref[...], w_ref[...],
                         preferred_element_type=jnp.float32) + b_ref[...]


def _proj(x2d, W, bias2d):
    return pl.pallas_call(
        _proj_kernel,
        grid=(B * L // MT,),
        in_specs=[
            pl.BlockSpec((MT, D), lambda m: (m, 0)),
            pl.BlockSpec((D, D), lambda m: (0, 0)),
            pl.BlockSpec((1, D), lambda m: (0, 0)),
        ],
        out_specs=pl.BlockSpec((MT, D), lambda m: (m, 0)),
        out_shape=jax.ShapeDtypeStruct((B * L, D), jnp.float32),
    )(x2d, W, bias2d)


def kernel(queries, keys, values, Wq, bq, Wk, bk, Wv, bv, Wo, bo):
    f32 = jnp.float32
    bq2, bk2 = bq.reshape(1, D), bk.reshape(1, D)

    NF = 1152
    t = jnp.arange(L, dtype=jnp.int32)
    p = (t[:, None] * t[None, :]) % L
    ang = p.astype(f32) * f32(2.0 * math.pi / L)
    Ctab = jnp.cos(ang)
    Stab = jnp.sin(ang)
    fidx = jnp.arange(NF)
    u = jnp.where((fidx == 0) | (fidx == L // 2), 1.0,
                  jnp.where(fidx < L // 2, 2.0, 0.0)).astype(f32)
    Cf = Ctab[:, :NF]
    Sf = Stab[:, :NF]
    Cu = Ctab[:NF, :] * u[:, None]
    Su = Stab[:NF, :] * u[:, None]

    wqh, wkh, bqh, bkh = pl.pallas_call(
        _red_kernel,
        grid=(4,),
        in_specs=[
            pl.BlockSpec((D // 4, D), lambda i: (i, 0)),
            pl.BlockSpec((D // 4, D), lambda i: (i, 0)),
            pl.BlockSpec((1, D), lambda i: (0, 0)),
            pl.BlockSpec((1, D), lambda i: (0, 0)),
        ],
        out_specs=[
            pl.BlockSpec((D // 4, H), lambda i: (i, 0)),
            pl.BlockSpec((D // 4, H), lambda i: (i, 0)),
            pl.BlockSpec((1, H), lambda i: (0, 0)),
            pl.BlockSpec((1, H), lambda i: (0, 0)),
        ],
        out_shape=[
            jax.ShapeDtypeStruct((D, H), f32),
            jax.ShapeDtypeStruct((D, H), f32),
            jax.ShapeDtypeStruct((1, H), f32),
            jax.ShapeDtypeStruct((1, H), f32),
        ],
    )(Wq, Wk, bq2, bk2)

    qm, km = pl.pallas_call(
        _means_kernel,
        grid=(B, L // MT),
        in_specs=[
            pl.BlockSpec((1, MT, D), lambda b, m: (b, m, 0)),
            pl.BlockSpec((1, MT, D), lambda b, m: (b, m, 0)),
            pl.BlockSpec((D, H), lambda b, m: (0, 0)),
            pl.BlockSpec((D, H), lambda b, m: (0, 0)),
            pl.BlockSpec((1, H), lambda b, m: (0, 0)),
            pl.BlockSpec((1, H), lambda b, m: (0, 0)),
        ],
        out_specs=[
            pl.BlockSpec((1, MT, H), lambda b, m: (b, m, 0)),
            pl.BlockSpec((1, MT, H), lambda b, m: (b, m, 0)),
        ],
        out_shape=[
            jax.ShapeDtypeStruct((B, L, H), f32),
            jax.ShapeDtypeStruct((B, L, H), f32),
        ],
    )(queries, keys, wqh, wkh, bqh, bkh)

    qmT = qm.transpose(0, 2, 1).reshape(B * H, L)
    kmT = km.transpose(0, 2, 1).reshape(B * H, L)

    FT = 384
    DT = 512
    pr, pi = pl.pallas_call(
        _spectrum_kernel,
        grid=(NF // FT,),
        in_specs=[
            pl.BlockSpec((B * H, L), lambda i: (0, 0)),
            pl.BlockSpec((B * H, L), lambda i: (0, 0)),
            pl.BlockSpec((L, FT), lambda i: (0, i)),
            pl.BlockSpec((L, FT), lambda i: (0, i)),
        ],
        out_specs=[
            pl.BlockSpec((B * H, FT), lambda i: (0, i)),
            pl.BlockSpec((B * H, FT), lambda i: (0, i)),
        ],
        out_shape=[
            jax.ShapeDtypeStruct((B * H, NF), f32),
            jax.ShapeDtypeStruct((B * H, NF), f32),
        ],
    )(qmT, kmT, Cf, Sf)

    corr = pl.pallas_call(
        _idft_kernel,
        grid=(L // DT,),
        in_specs=[
            pl.BlockSpec((B * H, NF), lambda i: (0, 0)),
            pl.BlockSpec((B * H, NF), lambda i: (0, 0)),
            pl.BlockSpec((NF, DT), lambda i: (0, i)),
            pl.BlockSpec((NF, DT), lambda i: (0, i)),
        ],
        out_specs=pl.BlockSpec((B * H, DT), lambda i: (0, i)),
        out_shape=jax.ShapeDtypeStruct((B * H, L), f32),
    )(pr, pi, Cu, Su)

    sc_topk = pl.kernel(
        _sc_topk_body,
        out_type=[
            jax.ShapeDtypeStruct((B * H * 16,), jnp.int32),
            jax.ShapeDtypeStruct((B * H * 16,), f32),
        ],
        mesh=plsc.VectorSubcoreMesh(core_axis_name="c", subcore_axis_name="s"),
        scratch_types=[
            pltpu.VMEM((L,), f32),
            pltpu.VMEM((32,), f32),
            pltpu.VMEM((32,), jnp.int32),
            pltpu.VMEM((16,), jnp.int32),
            pltpu.VMEM((16,), f32),
        ],
    )
    d16, w16 = sc_topk(corr.reshape(-1))
    delays = d16.reshape(B * H, 16)[:, :TOPK]
    wts = w16.reshape(B * H, 16)[:, :TOPK]

    v = _proj(values.reshape(B * L, D), Wv, bv.reshape(1, D)).reshape(B, L, D)

    ctx = pl.pallas_call(
        _roll_kernel,
        grid=(B, H),
        in_specs=[
            pl.BlockSpec(memory_space=pltpu.SMEM),
            pl.BlockSpec(memory_space=pltpu.SMEM),
            pl.BlockSpec((1, L, DK), lambda b, h: (b, 0, h)),
        ],
        out_specs=pl.BlockSpec((1, L, DK), lambda b, h: (b, 0, h)),
        out_shape=jax.ShapeDtypeStruct((B, L, D), f32),
        scratch_shapes=[pltpu.VMEM((2 * L, DK), f32)],
    )(delays.reshape(-1), wts.reshape(-1), v)

    out = _proj(ctx.reshape(B * L, D), Wo, bo.reshape(1, D))
    return out.reshape(B, L, D)
